# baseline (speedup 1.0000x reference)
import functools

import jax
import jax.numpy as jnp
from jax import lax
from jax.experimental import pallas as pl
from jax.experimental.pallas import tpu as pltpu
from jax.experimental.pallas import tpu_sc as plsc

MINPOS = -32
NBIN = 65
B = 2
L = 512
D = 128
NC = 2
NS = 16
NW = NC * NS
PAIRS = B * L
PAIRS_PER_W = PAIRS // NW
CHUNK = 128
NCHUNK = L // CHUNK


def _sc_body(x_hbm, idx_hbm, tab_hbm, out_hbm, idx_v, ib_v, xbuf, ebuf, tab_sp):
    cid = lax.axis_index("c")
    sid = lax.axis_index("s")
    wid = sid * NC + cid

    @pl.when(sid == 0)
    def _():
        pltpu.sync_copy(tab_hbm, tab_sp)

    pltpu.sync_copy(idx_hbm, idx_v)
    plsc.subcore_barrier()

    def chunk_body(t, _):
        pair = wid * PAIRS_PER_W + (t // NCHUNK)
        c = t % NCHUNK
        b = pair // L
        jbase = b * L + c * CHUNK
        row0 = pair * L + c * CHUNK

        pltpu.sync_copy(x_hbm.at[pl.ds(row0, CHUNK)], xbuf)

        vi = plsc.load_gather(idx_v, [jnp.full((16,), pair, jnp.int32)])
        for g in range(CHUNK // 16):
            jv = idx_v[pl.ds(jbase + g * 16, 16)]
            ib_v[pl.ds(g * 16, 16)] = jnp.clip(jv - vi - MINPOS, 0, NBIN - 1)

        pltpu.sync_copy(tab_sp.at[ib_v], ebuf)

        def add_row(r, _):
            for kk in range(D // 16):
                sl = pl.ds(kk * 16, 16)
                xbuf[r, sl] = xbuf[r, sl] + ebuf[r, sl]
            return _

        lax.fori_loop(0, CHUNK, add_row, 0)

        pltpu.sync_copy(xbuf, out_hbm.at[pl.ds(row0, CHUNK)])
        return _

    lax.fori_loop(0, PAIRS_PER_W * NCHUNK, chunk_body, 0)


def kernel(x, idx, emb_table):
    idx32 = idx.astype(jnp.int32).reshape(B * L)
    x_flat = x.reshape(B * L * L, D)
    mesh = plsc.VectorSubcoreMesh(core_axis_name="c", subcore_axis_name="s")
    out = pl.kernel(
        _sc_body,
        out_type=jax.ShapeDtypeStruct((B * L * L, D), jnp.float32),
        mesh=mesh,
        compiler_params=pltpu.CompilerParams(needs_layout_passes=False),
        scratch_types=[
            pltpu.VMEM((B * L,), jnp.int32),
            pltpu.VMEM((CHUNK,), jnp.int32),
            pltpu.VMEM((CHUNK, D), jnp.float32),
            pltpu.VMEM((CHUNK, D), jnp.float32),
            pltpu.VMEM_SHARED((NBIN, D), jnp.float32),
        ],
    )(x_flat, idx32, emb_table)
    return out.reshape(B, L, L, D)


# speedup vs baseline: 1.3072x; 1.3072x over previous
import functools

import jax
import jax.numpy as jnp
from jax import lax
from jax.experimental import pallas as pl
from jax.experimental.pallas import tpu as pltpu
from jax.experimental.pallas import tpu_sc as plsc

MINPOS = -32
NBIN = 65
B = 2
L = 512
D = 128
NC = 2
NS = 16
NW = NC * NS
PAIRS = B * L
PAIRS_PER_W = PAIRS // NW
CHUNK = 128
NCHUNK = L // CHUNK


def _sc_body(x_hbm, idx_hbm, tab_hbm, out_hbm, idx_v, ib_v, xbuf, ebuf, tab_sp):
    cid = lax.axis_index("c")
    sid = lax.axis_index("s")
    wid = sid * NC + cid

    @pl.when(sid == 0)
    def _():
        pltpu.sync_copy(tab_hbm, tab_sp)

    pltpu.sync_copy(idx_hbm, idx_v)
    plsc.subcore_barrier()

    def chunk_body(t, _):
        pair = wid * PAIRS_PER_W + (t // NCHUNK)
        c = t % NCHUNK
        b = pair // L
        jbase = b * L + c * CHUNK
        row0 = pair * L + c * CHUNK

        pltpu.sync_copy(x_hbm.at[pl.ds(row0, CHUNK)], xbuf)

        vi = plsc.load_gather(idx_v, [jnp.full((16,), pair, jnp.int32)])
        for g in range(CHUNK // 16):
            jv = idx_v[pl.ds(jbase + g * 16, 16)]
            ib_v[pl.ds(g * 16, 16)] = jnp.clip(jv - vi - MINPOS, 0, NBIN - 1)

        pltpu.sync_copy(tab_sp.at[ib_v], xbuf, add=True)

        pltpu.sync_copy(xbuf, out_hbm.at[pl.ds(row0, CHUNK)])
        return _

    lax.fori_loop(0, PAIRS_PER_W * NCHUNK, chunk_body, 0)


def kernel(x, idx, emb_table):
    idx32 = idx.astype(jnp.int32).reshape(B * L)
    x_flat = x.reshape(B * L * L, D)
    mesh = plsc.VectorSubcoreMesh(core_axis_name="c", subcore_axis_name="s")
    out = pl.kernel(
        _sc_body,
        out_type=jax.ShapeDtypeStruct((B * L * L, D), jnp.float32),
        mesh=mesh,
        compiler_params=pltpu.CompilerParams(needs_layout_passes=False),
        scratch_types=[
            pltpu.VMEM((B * L,), jnp.int32),
            pltpu.VMEM((CHUNK,), jnp.int32),
            pltpu.VMEM((CHUNK, D), jnp.float32),
            pltpu.VMEM((CHUNK, D), jnp.float32),
            pltpu.VMEM_SHARED((NBIN, D), jnp.float32),
        ],
    )(x_flat, idx32, emb_table)
    return out.reshape(B, L, L, D)


# speedup vs baseline: 2.2574x; 1.7269x over previous
import functools

import jax
import jax.numpy as jnp
from jax import lax
from jax.experimental import pallas as pl
from jax.experimental.pallas import tpu as pltpu
from jax.experimental.pallas import tpu_sc as plsc

MINPOS = -32
NBIN = 65
B = 2
L = 512
D = 128
NC = 2
NS = 16
NW = NC * NS
PAIRS = B * L
PAIRS_PER_W = PAIRS // NW
CHUNK = 128
NCHUNK = L // CHUNK


def _sc_body(x_hbm, idx_hbm, tab_hbm, out_hbm, idx_v, ib_v, xbuf, sem_in, tab_sp):
    cid = lax.axis_index("c")
    sid = lax.axis_index("s")
    wid = sid * NC + cid

    @pl.when(sid == 0)
    def _():
        pltpu.sync_copy(tab_hbm, tab_sp)

    pltpu.sync_copy(idx_hbm, idx_v)
    plsc.subcore_barrier()

    nchunks = PAIRS_PER_W * NCHUNK

    def chunk_row0(t):
        pair = wid * PAIRS_PER_W + (t // NCHUNK)
        c = t % NCHUNK
        return pair, pair * L + c * CHUNK

    def stage_in(t, k):
        pair, row0 = chunk_row0(t)
        b = pair // L
        jbase = b * L + (t % NCHUNK) * CHUNK
        vi = plsc.load_gather(idx_v, [jnp.full((16,), pair, jnp.int32)])
        for g in range(CHUNK // 16):
            jv = idx_v[pl.ds(jbase + g * 16, 16)]
            ib_v[k, pl.ds(g * 16, 16)] = jnp.clip(jv - vi - MINPOS, 0, NBIN - 1)
        pltpu.async_copy(x_hbm.at[pl.ds(row0, CHUNK)], xbuf.at[k], sem_in.at[k])

    stage_in(0, 0)

    def chunk_body(it, _):
        for k in (0, 1):
            t = 2 * it + k

            @pl.when(t + 1 < nchunks)
            def _():
                stage_in(t + 1, k ^ 1)

            _, row0 = chunk_row0(t)
            pltpu.make_async_copy(
                x_hbm.at[pl.ds(row0, CHUNK)], xbuf.at[k], sem_in.at[k]
            ).wait()
            pltpu.sync_copy(tab_sp.at[ib_v.at[k]], xbuf.at[k], add=True)
            pltpu.sync_copy(xbuf.at[k], out_hbm.at[pl.ds(row0, CHUNK)])
        return _

    lax.fori_loop(0, nchunks // 2, chunk_body, 0)


def kernel(x, idx, emb_table):
    idx32 = idx.astype(jnp.int32).reshape(B * L)
    x_flat = x.reshape(B * L * L, D)
    mesh = plsc.VectorSubcoreMesh(core_axis_name="c", subcore_axis_name="s")
    out = pl.kernel(
        _sc_body,
        out_type=jax.ShapeDtypeStruct((B * L * L, D), jnp.float32),
        mesh=mesh,
        compiler_params=pltpu.CompilerParams(needs_layout_passes=False),
        scratch_types=[
            pltpu.VMEM((B * L,), jnp.int32),
            pltpu.VMEM((2, CHUNK), jnp.int32),
            pltpu.VMEM((2, CHUNK, D), jnp.float32),
            pltpu.SemaphoreType.DMA((2,)),
            pltpu.VMEM_SHARED((NBIN, D), jnp.float32),
        ],
    )(x_flat, idx32, emb_table)
    return out.reshape(B, L, L, D)
